# baseline (speedup 1.0000x reference)
import functools

import jax
import jax.numpy as jnp
from jax import lax
from jax.experimental import pallas as pl
from jax.experimental.pallas import tpu as pltpu
from jax.experimental.pallas import tpu_sc as plsc

N = 100000
D = 256

NT = 80000
NSC = N - NT

BLOCK = 2000
GRID = NT // BLOCK


def _tc_body(u_ref, v_ref, o_ref):
    prod = u_ref[...] * v_ref[...]
    ones = jnp.ones((1, D), jnp.float32)
    s = jax.lax.dot_general(
        ones, prod, (((1,), (1,)), ((), ())),
        preferred_element_type=jnp.float32,
    )
    o_ref[...] = s.reshape(1, 1, BLOCK)


def _tc_part(gu, gi):
    out3 = pl.pallas_call(
        _tc_body,
        grid=(GRID,),
        in_specs=[
            pl.BlockSpec((BLOCK, D), lambda i: (i, 0)),
            pl.BlockSpec((BLOCK, D), lambda i: (i, 0)),
        ],
        out_specs=pl.BlockSpec((1, 1, BLOCK), lambda i: (i, 0, 0)),
        out_shape=jax.ShapeDtypeStruct((GRID, 1, BLOCK), jnp.float32),
    )(gu, gi)
    return out3.reshape(NT)


C = 80
CH0 = NT // C
NCHUNK = NSC // C
NW = 32
TPW = (NCHUNK + NW - 1) // NW
BASE_CH = NCHUNK // NW
TRIPS = (TPW + 1) // 2

_mesh = plsc.VectorSubcoreMesh(core_axis_name="c", subcore_axis_name="s")


@functools.partial(
    pl.kernel,
    mesh=_mesh,
    out_type=jax.ShapeDtypeStruct((NSC,), jnp.float32),
    scratch_types=[
        pltpu.VMEM((C, D), jnp.float32),
        pltpu.VMEM((C, D), jnp.float32),
        pltpu.VMEM((C, D), jnp.float32),
        pltpu.VMEM((C, D), jnp.float32),
        pltpu.VMEM((TPW * C + 16,), jnp.float32),
        pltpu.SemaphoreType.DMA,
        pltpu.SemaphoreType.DMA,
        pltpu.SemaphoreType.DMA,
    ],
    compiler_params=pltpu.CompilerParams(needs_layout_passes=False),
)
def _sc_rowdot(gu_hbm, gi_hbm, out_hbm, u0, v0, u1, v1, o_st, s0, s1, so):
    nc = 2
    wid = lax.axis_index("s") * nc + lax.axis_index("c")
    c0 = (wid * NCHUNK) // NW
    c1 = ((wid + 1) * NCHUNK) // NW
    my_n = c1 - c0

    bufs = ((u0, v0, s0), (u1, v1, s1))

    def issue(cid, b):
        u_b, v_b, s_b = bufs[b]
        base = (CH0 + cid) * C
        pltpu.async_copy(gu_hbm.at[pl.ds(base, C), :], u_b, s_b)
        pltpu.async_copy(gi_hbm.at[pl.ds(base, C), :], v_b, s_b)

    def drain(cid, b):
        u_b, v_b, s_b = bufs[b]
        base = (CH0 + cid) * C
        pltpu.make_async_copy(gu_hbm.at[pl.ds(base, C), :], u_b, s_b).wait()
        pltpu.make_async_copy(gi_hbm.at[pl.ds(base, C), :], v_b, s_b).wait()

    lane15 = lax.iota(jnp.int32, 16) == 15

    def compute(t, b):
        u_b, v_b, _ = bufs[b]

        def row_body(r, carry):
            accs = []
            for j in range(16):
                accs.append(
                    u_b[r, pl.ds(16 * j, 16)] * v_b[r, pl.ds(16 * j, 16)]
                )
            while len(accs) > 1:
                accs = [x + y for x, y in zip(accs[::2], accs[1::2])]
            tot = plsc.cumsum(accs[0])
            plsc.store_compressed(
                o_st.at[pl.ds(t * C + r, 16)], tot, mask=lane15
            )
            return carry

        lax.fori_loop(0, C, row_body, 0)

    issue(c0, 0)

    def trip_body(trip, carry):
        for b in range(2):
            t = 2 * trip + b
            cid = c0 + t

            @pl.when(cid < c1)
            def _():
                @pl.when(cid + 1 < c1)
                def _():
                    issue(cid + 1, 1 - b)

                drain(cid, b)
                compute(t, b)

        return carry

    lax.fori_loop(0, TRIPS, trip_body, 0)

    pltpu.async_copy(
        o_st.at[pl.ds(0, BASE_CH * C)],
        out_hbm.at[pl.ds(c0 * C, BASE_CH * C)],
        so,
    ).wait()

    @pl.when(my_n > BASE_CH)
    def _():
        pltpu.async_copy(
            o_st.at[pl.ds(BASE_CH * C, C)],
            out_hbm.at[pl.ds((c0 + BASE_CH) * C, C)],
            so,
        ).wait()


def kernel(gu, gi):
    sc_out = _sc_rowdot(gu, gi)
    tc_out = _tc_part(gu, gi)
    return jnp.concatenate([tc_out, sc_out])


# speedup vs baseline: 1.0588x; 1.0588x over previous
import functools

import jax
import jax.numpy as jnp
from jax import lax
from jax.experimental import pallas as pl
from jax.experimental.pallas import tpu as pltpu
from jax.experimental.pallas import tpu_sc as plsc

N = 100000
D = 256

NT = 80000
NSC = N - NT

BLOCK = 4000
GRID = NT // BLOCK


def _tc_body(u_ref, v_ref, o_ref):
    prod = u_ref[...] * v_ref[...]
    ones = jnp.ones((1, D), jnp.float32)
    s = jax.lax.dot_general(
        ones, prod, (((1,), (1,)), ((), ())),
        preferred_element_type=jnp.float32,
    )
    o_ref[...] = s.reshape(1, 1, BLOCK)


def _tc_part(gu, gi):
    out3 = pl.pallas_call(
        _tc_body,
        grid=(GRID,),
        in_specs=[
            pl.BlockSpec((BLOCK, D), lambda i: (i, 0)),
            pl.BlockSpec((BLOCK, D), lambda i: (i, 0)),
        ],
        out_specs=pl.BlockSpec((1, 1, BLOCK), lambda i: (i, 0, 0)),
        out_shape=jax.ShapeDtypeStruct((GRID, 1, BLOCK), jnp.float32),
    )(gu, gi)
    return out3.reshape(NT)


C = 80
CH0 = NT // C
NCHUNK = NSC // C
NW = 32
TPW = (NCHUNK + NW - 1) // NW
BASE_CH = NCHUNK // NW
TRIPS = (TPW + 1) // 2

_mesh = plsc.VectorSubcoreMesh(core_axis_name="c", subcore_axis_name="s")


@functools.partial(
    pl.kernel,
    mesh=_mesh,
    out_type=jax.ShapeDtypeStruct((NSC,), jnp.float32),
    scratch_types=[
        pltpu.VMEM((C, D), jnp.float32),
        pltpu.VMEM((C, D), jnp.float32),
        pltpu.VMEM((C, D), jnp.float32),
        pltpu.VMEM((C, D), jnp.float32),
        pltpu.VMEM((TPW * C + 16,), jnp.float32),
        pltpu.SemaphoreType.DMA,
        pltpu.SemaphoreType.DMA,
        pltpu.SemaphoreType.DMA,
    ],
    compiler_params=pltpu.CompilerParams(needs_layout_passes=False),
)
def _sc_rowdot(gu_hbm, gi_hbm, out_hbm, u0, v0, u1, v1, o_st, s0, s1, so):
    nc = 2
    wid = lax.axis_index("s") * nc + lax.axis_index("c")
    c0 = (wid * NCHUNK) // NW
    c1 = ((wid + 1) * NCHUNK) // NW
    my_n = c1 - c0

    bufs = ((u0, v0, s0), (u1, v1, s1))

    def issue(cid, b):
        u_b, v_b, s_b = bufs[b]
        base = (CH0 + cid) * C
        pltpu.async_copy(gu_hbm.at[pl.ds(base, C), :], u_b, s_b)
        pltpu.async_copy(gi_hbm.at[pl.ds(base, C), :], v_b, s_b)

    def drain(cid, b):
        u_b, v_b, s_b = bufs[b]
        base = (CH0 + cid) * C
        pltpu.make_async_copy(gu_hbm.at[pl.ds(base, C), :], u_b, s_b).wait()
        pltpu.make_async_copy(gi_hbm.at[pl.ds(base, C), :], v_b, s_b).wait()

    lane15 = lax.iota(jnp.int32, 16) == 15

    def compute(t, b):
        u_b, v_b, _ = bufs[b]

        def row_body(r, carry):
            accs = []
            for j in range(16):
                accs.append(
                    u_b[r, pl.ds(16 * j, 16)] * v_b[r, pl.ds(16 * j, 16)]
                )
            while len(accs) > 1:
                accs = [x + y for x, y in zip(accs[::2], accs[1::2])]
            tot = plsc.cumsum(accs[0])
            plsc.store_compressed(
                o_st.at[pl.ds(t * C + r, 16)], tot, mask=lane15
            )
            return carry

        lax.fori_loop(0, C, row_body, 0)

    issue(c0, 0)

    def trip_body(trip, carry):
        for b in range(2):
            t = 2 * trip + b
            cid = c0 + t

            @pl.when(cid < c1)
            def _():
                @pl.when(cid + 1 < c1)
                def _():
                    issue(cid + 1, 1 - b)

                drain(cid, b)
                compute(t, b)

        return carry

    lax.fori_loop(0, TRIPS, trip_body, 0)

    pltpu.async_copy(
        o_st.at[pl.ds(0, BASE_CH * C)],
        out_hbm.at[pl.ds(c0 * C, BASE_CH * C)],
        so,
    ).wait()

    @pl.when(my_n > BASE_CH)
    def _():
        pltpu.async_copy(
            o_st.at[pl.ds(BASE_CH * C, C)],
            out_hbm.at[pl.ds((c0 + BASE_CH) * C, C)],
            so,
        ).wait()


def kernel(gu, gi):
    sc_out = _sc_rowdot(gu, gi)
    tc_out = _tc_part(gu, gi)
    return jnp.concatenate([tc_out, sc_out])


# speedup vs baseline: 1.0626x; 1.0035x over previous
import functools

import jax
import jax.numpy as jnp
from jax import lax
from jax.experimental import pallas as pl
from jax.experimental.pallas import tpu as pltpu
from jax.experimental.pallas import tpu_sc as plsc

N = 100000
D = 256

NT = 88000
NSC = N - NT

BLOCK = 4000
GRID = NT // BLOCK


def _tc_body(u_ref, v_ref, o_ref):
    prod = u_ref[...] * v_ref[...]
    ones = jnp.ones((1, D), jnp.float32)
    s = jax.lax.dot_general(
        ones, prod, (((1,), (1,)), ((), ())),
        preferred_element_type=jnp.float32,
    )
    o_ref[...] = s.reshape(1, 1, BLOCK)


def _tc_part(gu, gi):
    out3 = pl.pallas_call(
        _tc_body,
        grid=(GRID,),
        in_specs=[
            pl.BlockSpec((BLOCK, D), lambda i: (i, 0)),
            pl.BlockSpec((BLOCK, D), lambda i: (i, 0)),
        ],
        out_specs=pl.BlockSpec((1, 1, BLOCK), lambda i: (i, 0, 0)),
        out_shape=jax.ShapeDtypeStruct((GRID, 1, BLOCK), jnp.float32),
    )(gu, gi)
    return out3.reshape(NT)


C = 80
CH0 = NT // C
NCHUNK = NSC // C
NW = 32
TPW = (NCHUNK + NW - 1) // NW
BASE_CH = NCHUNK // NW
TRIPS = (TPW + 1) // 2

_mesh = plsc.VectorSubcoreMesh(core_axis_name="c", subcore_axis_name="s")


@functools.partial(
    pl.kernel,
    mesh=_mesh,
    out_type=jax.ShapeDtypeStruct((NSC,), jnp.float32),
    scratch_types=[
        pltpu.VMEM((C, D), jnp.float32),
        pltpu.VMEM((C, D), jnp.float32),
        pltpu.VMEM((C, D), jnp.float32),
        pltpu.VMEM((C, D), jnp.float32),
        pltpu.VMEM((TPW * C + 16,), jnp.float32),
        pltpu.SemaphoreType.DMA,
        pltpu.SemaphoreType.DMA,
        pltpu.SemaphoreType.DMA,
    ],
    compiler_params=pltpu.CompilerParams(needs_layout_passes=False),
)
def _sc_rowdot(gu_hbm, gi_hbm, out_hbm, u0, v0, u1, v1, o_st, s0, s1, so):
    nc = 2
    wid = lax.axis_index("s") * nc + lax.axis_index("c")
    c0 = (wid * NCHUNK) // NW
    c1 = ((wid + 1) * NCHUNK) // NW
    my_n = c1 - c0

    bufs = ((u0, v0, s0), (u1, v1, s1))

    def issue(cid, b):
        u_b, v_b, s_b = bufs[b]
        base = (CH0 + cid) * C
        pltpu.async_copy(gu_hbm.at[pl.ds(base, C), :], u_b, s_b)
        pltpu.async_copy(gi_hbm.at[pl.ds(base, C), :], v_b, s_b)

    def drain(cid, b):
        u_b, v_b, s_b = bufs[b]
        base = (CH0 + cid) * C
        pltpu.make_async_copy(gu_hbm.at[pl.ds(base, C), :], u_b, s_b).wait()
        pltpu.make_async_copy(gi_hbm.at[pl.ds(base, C), :], v_b, s_b).wait()

    lane15 = lax.iota(jnp.int32, 16) == 15

    def compute(t, b):
        u_b, v_b, _ = bufs[b]

        def row_body(r, carry):
            accs = []
            for j in range(16):
                accs.append(
                    u_b[r, pl.ds(16 * j, 16)] * v_b[r, pl.ds(16 * j, 16)]
                )
            while len(accs) > 1:
                accs = [x + y for x, y in zip(accs[::2], accs[1::2])]
            tot = plsc.cumsum(accs[0])
            plsc.store_compressed(
                o_st.at[pl.ds(t * C + r, 16)], tot, mask=lane15
            )
            return carry

        lax.fori_loop(0, C, row_body, 0)

    issue(c0, 0)

    def trip_body(trip, carry):
        for b in range(2):
            t = 2 * trip + b
            cid = c0 + t

            @pl.when(cid < c1)
            def _():
                @pl.when(cid + 1 < c1)
                def _():
                    issue(cid + 1, 1 - b)

                drain(cid, b)
                compute(t, b)

        return carry

    lax.fori_loop(0, TRIPS, trip_body, 0)

    pltpu.async_copy(
        o_st.at[pl.ds(0, BASE_CH * C)],
        out_hbm.at[pl.ds(c0 * C, BASE_CH * C)],
        so,
    ).wait()

    @pl.when(my_n > BASE_CH)
    def _():
        pltpu.async_copy(
            o_st.at[pl.ds(BASE_CH * C, C)],
            out_hbm.at[pl.ds((c0 + BASE_CH) * C, C)],
            so,
        ).wait()


def kernel(gu, gi):
    sc_out = _sc_rowdot(gu, gi)
    tc_out = _tc_part(gu, gi)
    return jnp.concatenate([tc_out, sc_out])


# speedup vs baseline: 1.0647x; 1.0020x over previous
import functools

import jax
import jax.numpy as jnp
from jax import lax
from jax.experimental import pallas as pl
from jax.experimental.pallas import tpu as pltpu
from jax.experimental.pallas import tpu_sc as plsc

N = 100000
D = 256

NT = 68000
NSC = N - NT

BLOCK = 4000
GRID = NT // BLOCK


def _tc_body(u_ref, v_ref, o_ref):
    prod = u_ref[...] * v_ref[...]
    ones = jnp.ones((1, D), jnp.float32)
    s = jax.lax.dot_general(
        ones, prod, (((1,), (1,)), ((), ())),
        preferred_element_type=jnp.float32,
    )
    o_ref[...] = s.reshape(1, 1, BLOCK)


def _tc_part(gu, gi):
    out3 = pl.pallas_call(
        _tc_body,
        grid=(GRID,),
        in_specs=[
            pl.BlockSpec((BLOCK, D), lambda i: (i, 0)),
            pl.BlockSpec((BLOCK, D), lambda i: (i, 0)),
        ],
        out_specs=pl.BlockSpec((1, 1, BLOCK), lambda i: (i, 0, 0)),
        out_shape=jax.ShapeDtypeStruct((GRID, 1, BLOCK), jnp.float32),
    )(gu, gi)
    return out3.reshape(NT)


C = 80
CH0 = NT // C
NCHUNK = NSC // C
NW = 32
TPW = (NCHUNK + NW - 1) // NW
BASE_CH = NCHUNK // NW
TRIPS = (TPW + 1) // 2

_mesh = plsc.VectorSubcoreMesh(core_axis_name="c", subcore_axis_name="s")


@functools.partial(
    pl.kernel,
    mesh=_mesh,
    out_type=jax.ShapeDtypeStruct((NSC,), jnp.float32),
    scratch_types=[
        pltpu.VMEM((C, D), jnp.float32),
        pltpu.VMEM((C, D), jnp.float32),
        pltpu.VMEM((C, D), jnp.float32),
        pltpu.VMEM((C, D), jnp.float32),
        pltpu.VMEM((TPW * C + 16,), jnp.float32),
        pltpu.SemaphoreType.DMA,
        pltpu.SemaphoreType.DMA,
        pltpu.SemaphoreType.DMA,
    ],
    compiler_params=pltpu.CompilerParams(needs_layout_passes=False),
)
def _sc_rowdot(gu_hbm, gi_hbm, out_hbm, u0, v0, u1, v1, o_st, s0, s1, so):
    nc = 2
    wid = lax.axis_index("s") * nc + lax.axis_index("c")
    c0 = (wid * NCHUNK) // NW
    c1 = ((wid + 1) * NCHUNK) // NW
    my_n = c1 - c0

    bufs = ((u0, v0, s0), (u1, v1, s1))

    def issue(cid, b):
        u_b, v_b, s_b = bufs[b]
        base = (CH0 + cid) * C
        pltpu.async_copy(gu_hbm.at[pl.ds(base, C), :], u_b, s_b)
        pltpu.async_copy(gi_hbm.at[pl.ds(base, C), :], v_b, s_b)

    def drain(cid, b):
        u_b, v_b, s_b = bufs[b]
        base = (CH0 + cid) * C
        pltpu.make_async_copy(gu_hbm.at[pl.ds(base, C), :], u_b, s_b).wait()
        pltpu.make_async_copy(gi_hbm.at[pl.ds(base, C), :], v_b, s_b).wait()

    lane15 = lax.iota(jnp.int32, 16) == 15

    def compute(t, b):
        u_b, v_b, _ = bufs[b]

        def row_body(r, carry):
            accs = []
            for j in range(16):
                accs.append(
                    u_b[r, pl.ds(16 * j, 16)] * v_b[r, pl.ds(16 * j, 16)]
                )
            while len(accs) > 1:
                accs = [x + y for x, y in zip(accs[::2], accs[1::2])]
            tot = plsc.cumsum(accs[0])
            plsc.store_compressed(
                o_st.at[pl.ds(t * C + r, 16)], tot, mask=lane15
            )
            return carry

        lax.fori_loop(0, C, row_body, 0)

    issue(c0, 0)

    def trip_body(trip, carry):
        for b in range(2):
            t = 2 * trip + b
            cid = c0 + t

            @pl.when(cid < c1)
            def _():
                @pl.when(cid + 1 < c1)
                def _():
                    issue(cid + 1, 1 - b)

                drain(cid, b)
                compute(t, b)

        return carry

    lax.fori_loop(0, TRIPS, trip_body, 0)

    pltpu.async_copy(
        o_st.at[pl.ds(0, BASE_CH * C)],
        out_hbm.at[pl.ds(c0 * C, BASE_CH * C)],
        so,
    ).wait()

    @pl.when(my_n > BASE_CH)
    def _():
        pltpu.async_copy(
            o_st.at[pl.ds(BASE_CH * C, C)],
            out_hbm.at[pl.ds((c0 + BASE_CH) * C, C)],
            so,
        ).wait()


def kernel(gu, gi):
    sc_out = _sc_rowdot(gu, gi)
    tc_out = _tc_part(gu, gi)
    return jnp.concatenate([tc_out, sc_out])
